# baseline (speedup 1.0000x reference)
import jax
import jax.numpy as jnp
from jax import lax
from jax.experimental import pallas as pl
from jax.experimental.pallas import tpu as pltpu
from jax.experimental.pallas import tpu_sc as plsc

B = 128
V = 32768
NMAT = 7
SCM = 3
NC = 2
NS = 16
NW = NC * NS
RPW = B // NW
UNROLL = 8
NBUF = 3
NEG = float("-inf")
RB = 8


def _lane(x, i):
    idx = jnp.full((16,), i, jnp.int32)
    return x.at[idx].get(mode="promise_in_bounds")


def _merge(p1, p2, q1, q2):
    n1 = jnp.maximum(p1, q1)
    n2 = jnp.maximum(jnp.minimum(p1, q1), jnp.maximum(p2, q2))
    return n1, n2


def _scan_row(rowbuf):
    neg = jnp.full((16,), NEG, jnp.float32)

    @plsc.parallel_loop(0, V, step=UNROLL * 16, unroll=2,
                        carry=(neg,) * (2 * UNROLL))
    def body(i, c):
        acc = list(c)
        for u in range(UNROLL):
            v = rowbuf[pl.ds(i + u * 16, 16)]
            p1, p2 = acc[2 * u], acc[2 * u + 1]
            acc[2 * u] = jnp.maximum(p1, v)
            acc[2 * u + 1] = jnp.maximum(p2, jnp.minimum(p1, v))
        return tuple(acc)

    c = body
    while len(c) > 2:
        c = tuple(_merge(c[i], c[i + 1], c[i + 2], c[i + 3])[k]
                  for i in range(0, len(c), 4) for k in (0, 1))
    return c[0], c[1]


def _sc_body(o1, o2, o3, tgts, out_hbm, pmax_hbm,
             rowbuf0, rowbuf1, rowbuf2, tvbuf, stage, outstage,
             shared, redbuf, sem0, sem1, sem2):
    cid = lax.axis_index("c")
    sid = lax.axis_index("s")
    w = lax.axis_index("s") * NC + lax.axis_index("c")
    iota = lax.iota(jnp.int32, 16)
    zeros_i = jnp.full((16,), 0, jnp.int32)
    os = (o1, o2, o3)
    bufs = (rowbuf0, rowbuf1, rowbuf2)
    sems = (sem0, sem1, sem2)

    nrows = SCM * RPW
    descs = [None] * nrows
    for k in range(NBUF - 1):
        mk, jk = divmod(k, RPW)
        descs[k] = pltpu.async_copy(os[mk].at[RPW * w + jk], bufs[k % NBUF],
                                    sems[k % NBUF])

    pltpu.sync_copy(tgts, tvbuf)

    rmall = jnp.full((16,), NEG, jnp.float32)
    mg = jnp.zeros((16,), jnp.float32)
    for k in range(nrows):
        m, j = divmod(k, RPW)
        b = RPW * w + j
        kn = k + NBUF - 1
        if kn < nrows:
            mn, jn = divmod(kn, RPW)
            descs[kn] = pltpu.async_copy(
                os[mn].at[RPW * w + jn], bufs[kn % NBUF], sems[kn % NBUF])
        descs[k].wait()
        rowbuf = bufs[k % NBUF]
        m1, m2 = _scan_row(rowbuf)
        s1, _ = plsc.sort_key_val(m1, m1)
        s2, _ = plsc.sort_key_val(m2, m2)
        top1 = _lane(s1, 15)
        top2 = jnp.maximum(_lane(s1, 14), _lane(s2, 15))
        tsplat = plsc.load_gather(tvbuf, [zeros_i + b])
        tv = plsc.load_gather(rowbuf, [tsplat])
        margin = jnp.where(tv == top1, top1 - top2,
                           jnp.zeros((16,), jnp.float32))
        mg = jnp.where(iota == j, margin, mg)
        rmall = jnp.where(iota == j, jnp.maximum(rmall, top1), rmall)
        if j == RPW - 1:
            outstage[pl.ds(m * 16, 16)] = mg
            mg = jnp.zeros((16,), jnp.float32)

    pltpu.sync_copy(outstage, out_hbm.at[pl.ds(w * SCM * 16, SCM * 16)])

    stage[...] = rmall
    pltpu.sync_copy(stage, shared.at[pl.ds(sid * 16, 16)])
    plsc.subcore_barrier()

    @pl.when(sid == 0)
    def _():
        pltpu.sync_copy(shared, redbuf)

        def rbody(i, acc):
            return jnp.maximum(acc, redbuf[pl.ds(i * 16, 16)])

        acc = lax.fori_loop(0, NS, rbody, jnp.full((16,), NEG, jnp.float32))
        s, _ = plsc.sort_key_val(acc, acc)
        stage[...] = _lane(s, 15)
        pltpu.sync_copy(stage, pmax_hbm.at[pl.ds(cid * 16, 16)])


_mesh = plsc.VectorSubcoreMesh(core_axis_name="c", subcore_axis_name="s")
_params = pltpu.CompilerParams(needs_layout_passes=False)

_sc_kernel = pl.kernel(
    _sc_body,
    out_type=[
        jax.ShapeDtypeStruct((NW * SCM * 16,), jnp.float32),
        jax.ShapeDtypeStruct((NC * 16,), jnp.float32),
    ],
    mesh=_mesh,
    compiler_params=_params,
    scratch_types=[
        pltpu.VMEM((V,), jnp.float32),
        pltpu.VMEM((V,), jnp.float32),
        pltpu.VMEM((V,), jnp.float32),
        pltpu.VMEM((B,), jnp.int32),
        pltpu.VMEM((16,), jnp.float32),
        pltpu.VMEM((SCM * 16,), jnp.float32),
        pltpu.VMEM_SHARED((NS * 16,), jnp.float32),
        pltpu.VMEM((NS * 16,), jnp.float32),
        pltpu.SemaphoreType.DMA,
        pltpu.SemaphoreType.DMA,
        pltpu.SemaphoreType.DMA,
    ],
)


def _tc_margins_body(t_ref, o4_ref, o5_ref, o6_ref, om_ref, mg_ref, bm_ref):
    cols = lax.broadcasted_iota(jnp.int32, (RB, V), 1)
    tcol = t_ref[...]
    tmask = cols == tcol
    neg = jnp.float32(NEG)

    margins = []
    tops = []
    for x_ref in (o4_ref, o5_ref, o6_ref, om_ref):
        x = x_ref[...]
        top1 = jnp.max(x, axis=1, keepdims=True)
        eq = x == top1
        cnt = jnp.sum(eq.astype(jnp.float32), axis=1, keepdims=True)
        second = jnp.max(jnp.where(eq, neg, x), axis=1, keepdims=True)
        top2 = jnp.where(cnt >= 2.0, top1, second)
        tv = jnp.max(jnp.where(tmask, x, neg), axis=1, keepdims=True)
        margins.append(jnp.where(tv == top1, top1 - top2, 0.0))
        tops.append(jnp.max(x))

    mcol = lax.broadcasted_iota(jnp.int32, (RB, 128), 1)
    out = jnp.where(mcol == 0, margins[0],
                    jnp.where(mcol == 1, margins[1],
                              jnp.where(mcol == 2, margins[2],
                                        jnp.where(mcol == 3, margins[3],
                                                  0.0))))
    mg_ref[...] = out
    bcol = lax.broadcasted_iota(jnp.int32, (1, 1, 128), 2)
    bm_ref[...] = jnp.where(bcol == 0, tops[0],
                            jnp.where(bcol == 1, tops[1],
                                      jnp.where(bcol == 2, tops[2], neg)))


_tc_margins = pl.pallas_call(
    _tc_margins_body,
    grid=(B // RB,),
    in_specs=[
        pl.BlockSpec((RB, 1), lambda i: (i, 0)),
        pl.BlockSpec((RB, V), lambda i: (i, 0)),
        pl.BlockSpec((RB, V), lambda i: (i, 0)),
        pl.BlockSpec((RB, V), lambda i: (i, 0)),
        pl.BlockSpec((RB, V), lambda i: (i, 0)),
    ],
    out_specs=[
        pl.BlockSpec((RB, 128), lambda i: (i, 0)),
        pl.BlockSpec((1, 1, 128), lambda i: (i, 0, 0)),
    ],
    out_shape=[
        jax.ShapeDtypeStruct((B, 128), jnp.float32),
        jax.ShapeDtypeStruct((B // RB, 1, 128), jnp.float32),
    ],
)


def _tc_combine_body(preds_ref, bm_ref, scp_ref, out_ref, gm_ref):
    preds = preds_ref[...]
    mx = jnp.max(preds, axis=0, keepdims=True)
    e = jnp.exp((preds - mx) * 0.5)
    s = jnp.sum(e, axis=0, keepdims=True)
    out_ref[...] = e / s
    g = jnp.maximum(jnp.max(bm_ref[...]), jnp.max(scp_ref[...]))
    gm_ref[...] = jnp.full((1, 128), g, jnp.float32)


_tc_combine = pl.pallas_call(
    _tc_combine_body,
    out_shape=[
        jax.ShapeDtypeStruct((NMAT, B), jnp.float32),
        jax.ShapeDtypeStruct((1, 128), jnp.float32),
    ],
)


@jax.jit
def _run(outputs1, outputs2, outputs3, outputs4, outputs5, outputs6, mimic,
         targets, n_test):
    tgt = targets.astype(jnp.int32)
    sc_out, sc_pmax = _sc_kernel(outputs1, outputs2, outputs3, tgt)
    tc_mg, tc_bm = _tc_margins(tgt.reshape(B, 1), outputs4, outputs5,
                               outputs6, mimic)

    scm = sc_out.reshape(NW, SCM, 16)[:, :, :RPW]
    scm = scm.transpose(0, 2, 1).reshape(B, SCM)
    preds = jnp.concatenate([scm, tc_mg[:, :NMAT - SCM]], axis=1).T
    scp = jnp.concatenate([sc_pmax,
                           jnp.full((96,), NEG, jnp.float32)]).reshape(1, 128)
    sm, gm = _tc_combine(preds, tc_bm.reshape(B // RB, 128), scp)

    valid = (jnp.arange(B) < n_test)[:, None]
    out_threshold = jnp.where(valid, sm.T, jnp.float32(1.0 / NMAT))
    return gm[0, 0], out_threshold


def kernel(outputs1, outputs2, outputs3, outputs4, outputs5, outputs6, mimic,
           targets, n_test):
    return _run(outputs1, outputs2, outputs3, outputs4, outputs5, outputs6,
                mimic, targets, n_test)


# speedup vs baseline: 1.0325x; 1.0325x over previous
import jax
import jax.numpy as jnp
from jax import lax
from jax.experimental import pallas as pl
from jax.experimental.pallas import tpu as pltpu
from jax.experimental.pallas import tpu_sc as plsc

B = 128
V = 32768
NMAT = 7
SCM = 4
NC = 2
NS = 16
NW = NC * NS
RPW = B // NW
UNROLL = 8
NBUF = 3
NEG = float("-inf")
RB = 8


def _lane(x, i):
    idx = jnp.full((16,), i, jnp.int32)
    return x.at[idx].get(mode="promise_in_bounds")


def _merge(p1, p2, q1, q2):
    n1 = jnp.maximum(p1, q1)
    n2 = jnp.maximum(jnp.minimum(p1, q1), jnp.maximum(p2, q2))
    return n1, n2


def _scan_row(rowbuf):
    neg = jnp.full((16,), NEG, jnp.float32)

    @plsc.parallel_loop(0, V, step=UNROLL * 16, unroll=2,
                        carry=(neg,) * (2 * UNROLL))
    def body(i, c):
        acc = list(c)
        for u in range(UNROLL):
            v = rowbuf[pl.ds(i + u * 16, 16)]
            p1, p2 = acc[2 * u], acc[2 * u + 1]
            acc[2 * u] = jnp.maximum(p1, v)
            acc[2 * u + 1] = jnp.maximum(p2, jnp.minimum(p1, v))
        return tuple(acc)

    c = body
    while len(c) > 2:
        c = tuple(_merge(c[i], c[i + 1], c[i + 2], c[i + 3])[k]
                  for i in range(0, len(c), 4) for k in (0, 1))
    return c[0], c[1]


def _sc_body(o1, o2, o3, o4, tgts, out_hbm, pmax_hbm,
             rowbuf0, rowbuf1, rowbuf2, tvbuf, stage, outstage,
             shared, redbuf, sem0, sem1, sem2):
    cid = lax.axis_index("c")
    sid = lax.axis_index("s")
    w = lax.axis_index("s") * NC + lax.axis_index("c")
    iota = lax.iota(jnp.int32, 16)
    zeros_i = jnp.full((16,), 0, jnp.int32)
    os = (o1, o2, o3, o4)
    bufs = (rowbuf0, rowbuf1, rowbuf2)
    sems = (sem0, sem1, sem2)

    nrows = SCM * RPW
    descs = [None] * nrows
    for k in range(NBUF - 1):
        mk, jk = divmod(k, RPW)
        descs[k] = pltpu.async_copy(os[mk].at[RPW * w + jk], bufs[k % NBUF],
                                    sems[k % NBUF])

    pltpu.sync_copy(tgts, tvbuf)

    rmall = jnp.full((16,), NEG, jnp.float32)
    mg = jnp.zeros((16,), jnp.float32)
    for k in range(nrows):
        m, j = divmod(k, RPW)
        b = RPW * w + j
        kn = k + NBUF - 1
        if kn < nrows:
            mn, jn = divmod(kn, RPW)
            descs[kn] = pltpu.async_copy(
                os[mn].at[RPW * w + jn], bufs[kn % NBUF], sems[kn % NBUF])
        descs[k].wait()
        rowbuf = bufs[k % NBUF]
        m1, m2 = _scan_row(rowbuf)
        s1, _ = plsc.sort_key_val(m1, m1)
        s2, _ = plsc.sort_key_val(m2, m2)
        top1 = _lane(s1, 15)
        top2 = jnp.maximum(_lane(s1, 14), _lane(s2, 15))
        tsplat = plsc.load_gather(tvbuf, [zeros_i + b])
        tv = plsc.load_gather(rowbuf, [tsplat])
        margin = jnp.where(tv == top1, top1 - top2,
                           jnp.zeros((16,), jnp.float32))
        mg = jnp.where(iota == j, margin, mg)
        rmall = jnp.where(iota == j, jnp.maximum(rmall, top1), rmall)
        if j == RPW - 1:
            outstage[pl.ds(m * 16, 16)] = mg
            mg = jnp.zeros((16,), jnp.float32)

    pltpu.sync_copy(outstage, out_hbm.at[pl.ds(w * SCM * 16, SCM * 16)])

    stage[...] = rmall
    pltpu.sync_copy(stage, shared.at[pl.ds(sid * 16, 16)])
    plsc.subcore_barrier()

    @pl.when(sid == 0)
    def _():
        pltpu.sync_copy(shared, redbuf)

        def rbody(i, acc):
            return jnp.maximum(acc, redbuf[pl.ds(i * 16, 16)])

        acc = lax.fori_loop(0, NS, rbody, jnp.full((16,), NEG, jnp.float32))
        s, _ = plsc.sort_key_val(acc, acc)
        stage[...] = _lane(s, 15)
        pltpu.sync_copy(stage, pmax_hbm.at[pl.ds(cid * 16, 16)])


_mesh = plsc.VectorSubcoreMesh(core_axis_name="c", subcore_axis_name="s")
_params = pltpu.CompilerParams(needs_layout_passes=False)

_sc_kernel = pl.kernel(
    _sc_body,
    out_type=[
        jax.ShapeDtypeStruct((NW * SCM * 16,), jnp.float32),
        jax.ShapeDtypeStruct((NC * 16,), jnp.float32),
    ],
    mesh=_mesh,
    compiler_params=_params,
    scratch_types=[
        pltpu.VMEM((V,), jnp.float32),
        pltpu.VMEM((V,), jnp.float32),
        pltpu.VMEM((V,), jnp.float32),
        pltpu.VMEM((B,), jnp.int32),
        pltpu.VMEM((16,), jnp.float32),
        pltpu.VMEM((SCM * 16,), jnp.float32),
        pltpu.VMEM_SHARED((NS * 16,), jnp.float32),
        pltpu.VMEM((NS * 16,), jnp.float32),
        pltpu.SemaphoreType.DMA,
        pltpu.SemaphoreType.DMA,
        pltpu.SemaphoreType.DMA,
    ],
)


def _tc_margins_body(t_ref, o5_ref, o6_ref, om_ref, mg_ref, bm_ref):
    cols = lax.broadcasted_iota(jnp.int32, (RB, V), 1)
    tcol = t_ref[...]
    tmask = cols == tcol
    neg = jnp.float32(NEG)

    margins = []
    tops = []
    for x_ref in (o5_ref, o6_ref, om_ref):
        x = x_ref[...]
        top1 = jnp.max(x, axis=1, keepdims=True)
        eq = x == top1
        cnt = jnp.sum(eq.astype(jnp.float32), axis=1, keepdims=True)
        second = jnp.max(jnp.where(eq, neg, x), axis=1, keepdims=True)
        top2 = jnp.where(cnt >= 2.0, top1, second)
        tv = jnp.max(jnp.where(tmask, x, neg), axis=1, keepdims=True)
        margins.append(jnp.where(tv == top1, top1 - top2, 0.0))
        tops.append(jnp.max(x))

    mcol = lax.broadcasted_iota(jnp.int32, (RB, 128), 1)
    out = jnp.where(mcol == 0, margins[0],
                    jnp.where(mcol == 1, margins[1],
                              jnp.where(mcol == 2, margins[2], 0.0)))
    mg_ref[...] = out
    bcol = lax.broadcasted_iota(jnp.int32, (1, 1, 128), 2)
    bm_ref[...] = jnp.where(bcol == 0, tops[0],
                            jnp.where(bcol == 1, tops[1], neg))


_tc_margins = pl.pallas_call(
    _tc_margins_body,
    grid=(B // RB,),
    in_specs=[
        pl.BlockSpec((RB, 1), lambda i: (i, 0)),
        pl.BlockSpec((RB, V), lambda i: (i, 0)),
        pl.BlockSpec((RB, V), lambda i: (i, 0)),
        pl.BlockSpec((RB, V), lambda i: (i, 0)),
    ],
    out_specs=[
        pl.BlockSpec((RB, 128), lambda i: (i, 0)),
        pl.BlockSpec((1, 1, 128), lambda i: (i, 0, 0)),
    ],
    out_shape=[
        jax.ShapeDtypeStruct((B, 128), jnp.float32),
        jax.ShapeDtypeStruct((B // RB, 1, 128), jnp.float32),
    ],
)


def _tc_combine_body(preds_ref, bm_ref, scp_ref, out_ref, gm_ref):
    preds = preds_ref[...]
    mx = jnp.max(preds, axis=0, keepdims=True)
    e = jnp.exp((preds - mx) * 0.5)
    s = jnp.sum(e, axis=0, keepdims=True)
    out_ref[...] = e / s
    g = jnp.maximum(jnp.max(bm_ref[...]), jnp.max(scp_ref[...]))
    gm_ref[...] = jnp.full((1, 128), g, jnp.float32)


_tc_combine = pl.pallas_call(
    _tc_combine_body,
    out_shape=[
        jax.ShapeDtypeStruct((NMAT, B), jnp.float32),
        jax.ShapeDtypeStruct((1, 128), jnp.float32),
    ],
)


@jax.jit
def _run(outputs1, outputs2, outputs3, outputs4, outputs5, outputs6, mimic,
         targets, n_test):
    tgt = targets.astype(jnp.int32)
    sc_out, sc_pmax = _sc_kernel(outputs1, outputs2, outputs3, outputs4, tgt)
    tc_mg, tc_bm = _tc_margins(tgt.reshape(B, 1), outputs5, outputs6, mimic)

    scm = sc_out.reshape(NW, SCM, 16)[:, :, :RPW]
    scm = scm.transpose(0, 2, 1).reshape(B, SCM)
    preds = jnp.concatenate([scm, tc_mg[:, :NMAT - SCM]], axis=1).T
    scp = jnp.concatenate([sc_pmax,
                           jnp.full((96,), NEG, jnp.float32)]).reshape(1, 128)
    sm, gm = _tc_combine(preds, tc_bm.reshape(B // RB, 128), scp)

    valid = (jnp.arange(B) < n_test)[:, None]
    out_threshold = jnp.where(valid, sm.T, jnp.float32(1.0 / NMAT))
    return gm[0, 0], out_threshold


def kernel(outputs1, outputs2, outputs3, outputs4, outputs5, outputs6, mimic,
           targets, n_test):
    return _run(outputs1, outputs2, outputs3, outputs4, outputs5, outputs6,
                mimic, targets, n_test)


# speedup vs baseline: 1.0443x; 1.0115x over previous
import jax
import jax.numpy as jnp
from jax import lax
from jax.experimental import pallas as pl
from jax.experimental.pallas import tpu as pltpu
from jax.experimental.pallas import tpu_sc as plsc

B = 128
V = 32768
NMAT = 7
SCM = 4
NC = 2
NS = 16
NW = NC * NS
RPW = B // NW
UNROLL = 8
NBUF = 3
NEG = float("-inf")
RB = 8


def _lane(x, i):
    idx = jnp.full((16,), i, jnp.int32)
    return x.at[idx].get(mode="promise_in_bounds")


def _merge(p1, p2, q1, q2):
    n1 = jnp.maximum(p1, q1)
    n2 = jnp.maximum(jnp.minimum(p1, q1), jnp.maximum(p2, q2))
    return n1, n2


def _scan_row(rowbuf):
    neg = jnp.full((16,), NEG, jnp.float32)

    @plsc.parallel_loop(0, V, step=UNROLL * 16, unroll=2,
                        carry=(neg,) * (2 * UNROLL))
    def body(i, c):
        acc = list(c)
        for u in range(UNROLL):
            v = rowbuf[pl.ds(i + u * 16, 16)]
            p1, p2 = acc[2 * u], acc[2 * u + 1]
            acc[2 * u] = jnp.maximum(p1, v)
            acc[2 * u + 1] = jnp.maximum(p2, jnp.minimum(p1, v))
        return tuple(acc)

    c = body
    while len(c) > 2:
        c = tuple(_merge(c[i], c[i + 1], c[i + 2], c[i + 3])[k]
                  for i in range(0, len(c), 4) for k in (0, 1))
    return c[0], c[1]


def _sc_body(o1, o2, o3, o4, tgts, out_hbm, pmax_hbm,
             rowbuf0, rowbuf1, rowbuf2, tvbuf, stage, outstage,
             shared, redbuf, sem0, sem1, sem2):
    cid = lax.axis_index("c")
    sid = lax.axis_index("s")
    w = lax.axis_index("s") * NC + lax.axis_index("c")
    iota = lax.iota(jnp.int32, 16)
    zeros_i = jnp.full((16,), 0, jnp.int32)
    os = (o1, o2, o3, o4)
    bufs = (rowbuf0, rowbuf1, rowbuf2)
    sems = (sem0, sem1, sem2)

    def _start_row(mk, jk, buf, sem):
        r = RPW * w + jk
        h = V // 2
        da = pltpu.async_copy(os[mk].at[r, pl.ds(0, h)], buf.at[pl.ds(0, h)],
                              sem)
        db = pltpu.async_copy(os[mk].at[r, pl.ds(h, h)], buf.at[pl.ds(h, h)],
                              sem)
        return (da, db)

    nrows = SCM * RPW
    descs = [None] * nrows
    for k in range(NBUF - 1):
        mk, jk = divmod(k, RPW)
        descs[k] = _start_row(mk, jk, bufs[k % NBUF], sems[k % NBUF])

    pltpu.sync_copy(tgts, tvbuf)

    rmall = jnp.full((16,), NEG, jnp.float32)
    mg = jnp.zeros((16,), jnp.float32)
    for k in range(nrows):
        m, j = divmod(k, RPW)
        b = RPW * w + j
        kn = k + NBUF - 1
        if kn < nrows:
            mn, jn = divmod(kn, RPW)
            descs[kn] = _start_row(mn, jn, bufs[kn % NBUF], sems[kn % NBUF])
        descs[k][0].wait()
        descs[k][1].wait()
        rowbuf = bufs[k % NBUF]
        m1, m2 = _scan_row(rowbuf)
        s1, _ = plsc.sort_key_val(m1, m1)
        s2, _ = plsc.sort_key_val(m2, m2)
        top1 = _lane(s1, 15)
        top2 = jnp.maximum(_lane(s1, 14), _lane(s2, 15))
        tsplat = plsc.load_gather(tvbuf, [zeros_i + b])
        tv = plsc.load_gather(rowbuf, [tsplat])
        margin = jnp.where(tv == top1, top1 - top2,
                           jnp.zeros((16,), jnp.float32))
        mg = jnp.where(iota == j, margin, mg)
        rmall = jnp.where(iota == j, jnp.maximum(rmall, top1), rmall)
        if j == RPW - 1:
            outstage[pl.ds(m * 16, 16)] = mg
            mg = jnp.zeros((16,), jnp.float32)

    pltpu.sync_copy(outstage, out_hbm.at[pl.ds(w * SCM * 16, SCM * 16)])

    stage[...] = rmall
    pltpu.sync_copy(stage, shared.at[pl.ds(sid * 16, 16)])
    plsc.subcore_barrier()

    @pl.when(sid == 0)
    def _():
        pltpu.sync_copy(shared, redbuf)

        def rbody(i, acc):
            return jnp.maximum(acc, redbuf[pl.ds(i * 16, 16)])

        acc = lax.fori_loop(0, NS, rbody, jnp.full((16,), NEG, jnp.float32))
        s, _ = plsc.sort_key_val(acc, acc)
        stage[...] = _lane(s, 15)
        pltpu.sync_copy(stage, pmax_hbm.at[pl.ds(cid * 16, 16)])


_mesh = plsc.VectorSubcoreMesh(core_axis_name="c", subcore_axis_name="s")
_params = pltpu.CompilerParams(needs_layout_passes=False)

_sc_kernel = pl.kernel(
    _sc_body,
    out_type=[
        jax.ShapeDtypeStruct((NW * SCM * 16,), jnp.float32),
        jax.ShapeDtypeStruct((NC * 16,), jnp.float32),
    ],
    mesh=_mesh,
    compiler_params=_params,
    scratch_types=[
        pltpu.VMEM((V,), jnp.float32),
        pltpu.VMEM((V,), jnp.float32),
        pltpu.VMEM((V,), jnp.float32),
        pltpu.VMEM((B,), jnp.int32),
        pltpu.VMEM((16,), jnp.float32),
        pltpu.VMEM((SCM * 16,), jnp.float32),
        pltpu.VMEM_SHARED((NS * 16,), jnp.float32),
        pltpu.VMEM((NS * 16,), jnp.float32),
        pltpu.SemaphoreType.DMA,
        pltpu.SemaphoreType.DMA,
        pltpu.SemaphoreType.DMA,
    ],
)


def _tc_margins_body(t_ref, o5_ref, o6_ref, om_ref, mg_ref, bm_ref):
    cols = lax.broadcasted_iota(jnp.int32, (RB, V), 1)
    tcol = t_ref[...]
    tmask = cols == tcol
    neg = jnp.float32(NEG)

    margins = []
    tops = []
    for x_ref in (o5_ref, o6_ref, om_ref):
        x = x_ref[...]
        top1 = jnp.max(x, axis=1, keepdims=True)
        eq = x == top1
        cnt = jnp.sum(eq.astype(jnp.float32), axis=1, keepdims=True)
        second = jnp.max(jnp.where(eq, neg, x), axis=1, keepdims=True)
        top2 = jnp.where(cnt >= 2.0, top1, second)
        tv = jnp.max(jnp.where(tmask, x, neg), axis=1, keepdims=True)
        margins.append(jnp.where(tv == top1, top1 - top2, 0.0))
        tops.append(jnp.max(x))

    mcol = lax.broadcasted_iota(jnp.int32, (RB, 128), 1)
    out = jnp.where(mcol == 0, margins[0],
                    jnp.where(mcol == 1, margins[1],
                              jnp.where(mcol == 2, margins[2], 0.0)))
    mg_ref[...] = out
    bcol = lax.broadcasted_iota(jnp.int32, (1, 1, 128), 2)
    bm_ref[...] = jnp.where(bcol == 0, tops[0],
                            jnp.where(bcol == 1, tops[1], neg))


_tc_margins = pl.pallas_call(
    _tc_margins_body,
    grid=(B // RB,),
    in_specs=[
        pl.BlockSpec((RB, 1), lambda i: (i, 0)),
        pl.BlockSpec((RB, V), lambda i: (i, 0)),
        pl.BlockSpec((RB, V), lambda i: (i, 0)),
        pl.BlockSpec((RB, V), lambda i: (i, 0)),
    ],
    out_specs=[
        pl.BlockSpec((RB, 128), lambda i: (i, 0)),
        pl.BlockSpec((1, 1, 128), lambda i: (i, 0, 0)),
    ],
    out_shape=[
        jax.ShapeDtypeStruct((B, 128), jnp.float32),
        jax.ShapeDtypeStruct((B // RB, 1, 128), jnp.float32),
    ],
)


def _tc_combine_body(preds_ref, bm_ref, scp_ref, out_ref, gm_ref):
    preds = preds_ref[...]
    mx = jnp.max(preds, axis=0, keepdims=True)
    e = jnp.exp((preds - mx) * 0.5)
    s = jnp.sum(e, axis=0, keepdims=True)
    out_ref[...] = e / s
    g = jnp.maximum(jnp.max(bm_ref[...]), jnp.max(scp_ref[...]))
    gm_ref[...] = jnp.full((1, 128), g, jnp.float32)


_tc_combine = pl.pallas_call(
    _tc_combine_body,
    out_shape=[
        jax.ShapeDtypeStruct((NMAT, B), jnp.float32),
        jax.ShapeDtypeStruct((1, 128), jnp.float32),
    ],
)


@jax.jit
def _run(outputs1, outputs2, outputs3, outputs4, outputs5, outputs6, mimic,
         targets, n_test):
    tgt = targets.astype(jnp.int32)
    sc_out, sc_pmax = _sc_kernel(outputs1, outputs2, outputs3, outputs4, tgt)
    tc_mg, tc_bm = _tc_margins(tgt.reshape(B, 1), outputs5, outputs6, mimic)

    scm = sc_out.reshape(NW, SCM, 16)[:, :, :RPW]
    scm = scm.transpose(0, 2, 1).reshape(B, SCM)
    preds = jnp.concatenate([scm, tc_mg[:, :NMAT - SCM]], axis=1).T
    scp = jnp.concatenate([sc_pmax,
                           jnp.full((96,), NEG, jnp.float32)]).reshape(1, 128)
    sm, gm = _tc_combine(preds, tc_bm.reshape(B // RB, 128), scp)

    valid = (jnp.arange(B) < n_test)[:, None]
    out_threshold = jnp.where(valid, sm.T, jnp.float32(1.0 / NMAT))
    return gm[0, 0], out_threshold


def kernel(outputs1, outputs2, outputs3, outputs4, outputs5, outputs6, mimic,
           targets, n_test):
    return _run(outputs1, outputs2, outputs3, outputs4, outputs5, outputs6,
                mimic, targets, n_test)
